# baseline (speedup 1.0000x reference)
import jax
import jax.numpy as jnp
from jax.experimental import pallas as pl


SEQ_BLOCK = 256


def _add_kernel(x_ref, emb_ref, o_ref):
    o_ref[...] = x_ref[...] + emb_ref[...]


def kernel(x, emb):
    bs, sl, d = x.shape
    nsb = sl // SEQ_BLOCK
    return pl.pallas_call(
        _add_kernel,
        grid=(nsb,),
        in_specs=[
            pl.BlockSpec((bs, SEQ_BLOCK, d), lambda i: (0, i, 0)),
            pl.BlockSpec((SEQ_BLOCK, d), lambda i: (i, 0)),
        ],
        out_specs=pl.BlockSpec((bs, SEQ_BLOCK, d), lambda i: (0, i, 0)),
        out_shape=jax.ShapeDtypeStruct((bs, sl, d), x.dtype),
    )(x, emb)


# speedup vs baseline: 1.0316x; 1.0316x over previous
import jax
import jax.numpy as jnp
from jax.experimental import pallas as pl


SEQ_BLOCK = 1024


def _add_kernel(x_ref, emb_ref, o_ref):
    o_ref[...] = x_ref[...] + emb_ref[...]


def kernel(x, emb):
    bs, sl, d = x.shape
    nsb = sl // SEQ_BLOCK
    return pl.pallas_call(
        _add_kernel,
        grid=(nsb,),
        in_specs=[
            pl.BlockSpec((bs, SEQ_BLOCK, d), lambda i: (0, i, 0)),
            pl.BlockSpec((SEQ_BLOCK, d), lambda i: (i, 0)),
        ],
        out_specs=pl.BlockSpec((bs, SEQ_BLOCK, d), lambda i: (0, i, 0)),
        out_shape=jax.ShapeDtypeStruct((bs, sl, d), x.dtype),
    )(x, emb)
